# Initial kernel scaffold; baseline (speedup 1.0000x reference)
import jax
import jax.numpy as jnp
from jax.experimental import pallas as pl


def kernel(x, edge_index, spatial_coords, ln1_g, ln1_b, W_self1, b_self1, W_nei1, b_nei1, ln2_g, ln2_b, W_self2, b_self2, W_nei2, b_nei2, W_red, b_red):
    raise NotImplementedError("write your pallas kernel here")


# speedup vs baseline: 1.6297x; 1.6297x over previous
import functools

import jax
import jax.numpy as jnp
from jax import lax
from jax.experimental import pallas as pl
from jax.experimental.pallas import tpu as pltpu
from jax.experimental.pallas import tpu_sc as plsc

RADIUS = 1.0
ALPHA = 1.0
BETA1 = 0.5
BETA2 = 0.5
NEG_SLOPE = 0.01
LN_EPS = 1e-5

L = 16
NW = 32
GRP = 2


def _leaky(v):
    return jnp.where(v >= 0, v, NEG_SLOPE * v)


def _stage1_body(x_ref, crd_ref, g1_ref, b1_ref, wn1_ref, bn1_ref,
                 g2_ref, b2_ref, wn2_ref, bn2_ref, t_ref):
    xb = x_ref[:, :]
    d = xb.shape[1]
    mu = jnp.mean(xb, axis=1, keepdims=True)
    var = jnp.mean((xb - mu) ** 2, axis=1, keepdims=True)
    stdp = jnp.sqrt(var + LN_EPS)
    lnx = (xb - mu) / stdp
    ln1x = lnx * g1_ref[0, :] + b1_ref[0, :]
    ln2x = lnx * g2_ref[0, :] + b2_ref[0, :]
    s1 = lax.dot_general(ln1x, wn1_ref[:, :], (((1,), (1,)), ((), ())),
                         preferred_element_type=jnp.float32) + bn1_ref[0, :]
    s2 = lax.dot_general(ln2x, wn2_ref[:, :], (((1,), (1,)), ((), ())),
                         preferred_element_type=jnp.float32) + bn2_ref[0, :]
    nrows = xb.shape[0]
    tail = jnp.concatenate(
        [mu, stdp, crd_ref[:, :], jnp.zeros((nrows, d - 4), jnp.float32)],
        axis=1)
    t_ref[:, :] = jnp.concatenate([lnx, s1, s2, tail], axis=1)


def _rsqrt_nr(s):
    y = lax.bitcast_convert_type(
        jnp.int32(0x5F3759DF) - (lax.bitcast_convert_type(s, jnp.int32) >> 1),
        jnp.float32)
    for _ in range(3):
        y = y * (1.5 - 0.5 * s * y * y)
    return y


def _sc_body(deg, d, npw, t_hbm, col_hbm, crd_hbm, r_hbm,
             idx_v, rows_v, crd_v, out_v):
    w = lax.axis_index("c") * 16 + lax.axis_index("s")
    nbase = w * npw
    ebase = nbase * deg
    epw = npw * deg
    pltpu.sync_copy(col_hbm.at[pl.ds(ebase, epw)], idx_v)
    pltpu.sync_copy(crd_hbm.at[pl.ds(nbase * 8, npw * 8)],
                    crd_v.at[pl.ds(0, npw * 8)])

    zero16 = jnp.zeros((L,), jnp.float32)
    dscale = jnp.float32(-ALPHA / (RADIUS + 1e-8))
    c_mu = 3 * d

    def group_body(g, carry):
        pltpu.sync_copy(t_hbm.at[idx_v.at[pl.ds(g * (GRP * deg), GRP * deg)]],
                        rows_v)
        for t in range(GRP):
            nl = g * GRP + t
            own = crd_v[pl.ds(nl * 8, L)]
            cxn = jnp.full((L,), own[0], jnp.float32)
            cyn = jnp.full((L,), own[1], jnp.float32)
            for j in range(5 * d // L):
                out_v[t, pl.ds(j * L, L)] = zero16

            def k_body(k, kcarry):
                r = t * deg + k
                tail = rows_v[r, pl.ds(c_mu, L)]
                muk = jnp.full((L,), tail[0], jnp.float32)
                stk = jnp.full((L,), tail[1], jnp.float32)
                cx = jnp.full((L,), tail[2], jnp.float32)
                cy = jnp.full((L,), tail[3], jnp.float32)
                d2 = (cxn - cx) * (cxn - cx) + (cyn - cy) * (cyn - cy)
                s = jnp.maximum(d2, 1e-30)
                dist = s * _rsqrt_nr(s)
                dwk = jnp.exp(dist * dscale)
                for j in range(d // L):
                    nb = rows_v[r, pl.ds(j * L, L)]
                    s1 = rows_v[r, pl.ds(d + j * L, L)]
                    s2 = rows_v[r, pl.ds(2 * d + j * L, L)]
                    e1 = jnp.exp(s1 * dwk)
                    e2 = jnp.exp(s2 * dwk)
                    plsc.addupdate(out_v.at[t, pl.ds(j * L, L)], nb * stk + muk)
                    plsc.addupdate(out_v.at[t, pl.ds(d + j * L, L)], e1)
                    plsc.addupdate(out_v.at[t, pl.ds(2 * d + j * L, L)], e1 * nb)
                    plsc.addupdate(out_v.at[t, pl.ds(3 * d + j * L, L)], e2)
                    plsc.addupdate(out_v.at[t, pl.ds(4 * d + j * L, L)], e2 * nb)
                return kcarry

            lax.fori_loop(0, deg, k_body, 0)
        pltpu.sync_copy(out_v, r_hbm.at[pl.ds(nbase + g * GRP, GRP)])
        return carry

    lax.fori_loop(0, npw // GRP, group_body, 0)


def _stage3_body(r_ref, g1_ref, b1_ref, ws1_ref, bs1_ref,
                 g2_ref, b2_ref, ws2_ref, bs2_ref, wr_ref, br_ref, out_ref):
    d = ws1_ref.shape[0]
    rb = r_ref[:, :]
    agg = rb[:, 0:d]
    den1 = rb[:, d:2 * d]
    m1 = rb[:, 2 * d:3 * d]
    den2 = rb[:, 3 * d:4 * d]
    m2 = rb[:, 4 * d:5 * d]

    def ln(v, g, b):
        mu = jnp.mean(v, axis=1, keepdims=True)
        var = jnp.mean((v - mu) ** 2, axis=1, keepdims=True)
        return (v - mu) * lax.rsqrt(var + LN_EPS) * g + b

    t1 = ln(agg, g1_ref[0, :], b1_ref[0, :])
    ss1 = lax.dot_general(t1, ws1_ref[:, :], (((1,), (1,)), ((), ())),
                          preferred_element_type=jnp.float32) + bs1_ref[0, :]
    es1 = jnp.exp(ss1)
    dtot1 = es1 + den1
    num1 = g1_ref[0, :] * m1 + b1_ref[0, :] * den1
    x1 = _leaky((BETA1 * es1 * t1 + (1.0 - BETA1) * num1) / dtot1)

    t2 = ln(x1, g2_ref[0, :], b2_ref[0, :])
    ss2 = lax.dot_general(t2, ws2_ref[:, :], (((1,), (1,)), ((), ())),
                          preferred_element_type=jnp.float32) + bs2_ref[0, :]
    es2 = jnp.exp(ss2)
    dtot2 = es2 + den2
    num2 = g2_ref[0, :] * m2 + b2_ref[0, :] * den2
    x2 = _leaky((BETA2 * es2 * t2 + (1.0 - BETA2) * num2) / dtot2)

    out_ref[:, :] = lax.dot_general(x2, wr_ref[:, :], (((1,), (1,)), ((), ())),
                                    preferred_element_type=jnp.float32) + br_ref[0, :]


def kernel(x, edge_index, spatial_coords, ln1_g, ln1_b, W_self1, b_self1,
           W_nei1, b_nei1, ln2_g, ln2_b, W_self2, b_self2, W_nei2, b_nei2,
           W_red, b_red):
    n, d = x.shape
    deg = edge_index.shape[1] // n
    z = W_red.shape[0]
    tw = 4 * d
    row = lambda v: v.reshape(1, -1)

    b1blk = 1000
    t_tab = pl.pallas_call(
        _stage1_body,
        grid=(n // b1blk,),
        in_specs=[
            pl.BlockSpec((b1blk, d), lambda i: (i, 0)),
            pl.BlockSpec((b1blk, 2), lambda i: (i, 0)),
            pl.BlockSpec((1, d), lambda i: (0, 0)),
            pl.BlockSpec((1, d), lambda i: (0, 0)),
            pl.BlockSpec((d, d), lambda i: (0, 0)),
            pl.BlockSpec((1, d), lambda i: (0, 0)),
            pl.BlockSpec((1, d), lambda i: (0, 0)),
            pl.BlockSpec((1, d), lambda i: (0, 0)),
            pl.BlockSpec((d, d), lambda i: (0, 0)),
            pl.BlockSpec((1, d), lambda i: (0, 0)),
        ],
        out_specs=pl.BlockSpec((b1blk, tw), lambda i: (i, 0)),
        out_shape=jax.ShapeDtypeStruct((n, tw), jnp.float32),
    )(x, spatial_coords, row(ln1_g), row(ln1_b), W_nei1, row(b_nei1),
      row(ln2_g), row(ln2_b), W_nei2, row(b_nei2))

    npw = -(-n // (NW * 64)) * 64
    npad = NW * npw
    col = edge_index[1]
    col_pad = jnp.pad(col, (0, (npad - n) * deg))
    crd_pad = jnp.zeros((npad, 8), jnp.float32).at[:n, :2].set(
        spatial_coords).reshape(-1)

    mesh = plsc.VectorSubcoreMesh(core_axis_name="c", subcore_axis_name="s",
                                  num_cores=2, num_subcores=16)
    sc = pl.kernel(
        functools.partial(_sc_body, deg, d, npw),
        out_type=jax.ShapeDtypeStruct((npad, 5 * d), jnp.float32),
        mesh=mesh,
        scratch_types=[
            pltpu.VMEM((npw * deg,), jnp.int32),
            pltpu.VMEM((GRP * deg, tw), jnp.float32),
            pltpu.VMEM((npw * 8 + 8,), jnp.float32),
            pltpu.VMEM((GRP, 5 * d), jnp.float32),
        ],
    )
    r_acc = sc(t_tab, col_pad, crd_pad)

    b3blk = 1024
    out = pl.pallas_call(
        _stage3_body,
        grid=(npad // b3blk,),
        in_specs=[
            pl.BlockSpec((b3blk, 5 * d), lambda i: (i, 0)),
            pl.BlockSpec((1, d), lambda i: (0, 0)),
            pl.BlockSpec((1, d), lambda i: (0, 0)),
            pl.BlockSpec((d, d), lambda i: (0, 0)),
            pl.BlockSpec((1, d), lambda i: (0, 0)),
            pl.BlockSpec((1, d), lambda i: (0, 0)),
            pl.BlockSpec((1, d), lambda i: (0, 0)),
            pl.BlockSpec((d, d), lambda i: (0, 0)),
            pl.BlockSpec((1, d), lambda i: (0, 0)),
            pl.BlockSpec((z, d), lambda i: (0, 0)),
            pl.BlockSpec((1, z), lambda i: (0, 0)),
        ],
        out_specs=pl.BlockSpec((b3blk, z), lambda i: (i, 0)),
        out_shape=jax.ShapeDtypeStruct((npad, z), jnp.float32),
    )(r_acc, row(ln1_g), row(ln1_b), W_self1, row(b_self1),
      row(ln2_g), row(ln2_b), W_self2, row(b_self2), W_red, row(b_red))
    return out[:n]


# speedup vs baseline: 2.2018x; 1.3510x over previous
import functools

import jax
import jax.numpy as jnp
from jax import lax
from jax.experimental import pallas as pl
from jax.experimental.pallas import tpu as pltpu
from jax.experimental.pallas import tpu_sc as plsc

RADIUS = 1.0
ALPHA = 1.0
BETA1 = 0.5
BETA2 = 0.5
NEG_SLOPE = 0.01
LN_EPS = 1e-5

L = 16
NW = 32
GRP = 2


def _leaky(v):
    return jnp.where(v >= 0, v, NEG_SLOPE * v)


def _stage1_body(x_ref, crd_ref, g1_ref, b1_ref, wn1_ref, bn1_ref,
                 g2_ref, b2_ref, wn2_ref, bn2_ref, t_ref):
    xb = x_ref[:, :]
    d = xb.shape[1]
    mu = jnp.mean(xb, axis=1, keepdims=True)
    var = jnp.mean((xb - mu) ** 2, axis=1, keepdims=True)
    stdp = jnp.sqrt(var + LN_EPS)
    lnx = (xb - mu) / stdp
    ln1x = lnx * g1_ref[0, :] + b1_ref[0, :]
    ln2x = lnx * g2_ref[0, :] + b2_ref[0, :]
    s1 = lax.dot_general(ln1x, wn1_ref[:, :], (((1,), (1,)), ((), ())),
                         preferred_element_type=jnp.float32) + bn1_ref[0, :]
    s2 = lax.dot_general(ln2x, wn2_ref[:, :], (((1,), (1,)), ((), ())),
                         preferred_element_type=jnp.float32) + bn2_ref[0, :]
    nrows = xb.shape[0]
    tail = jnp.concatenate(
        [mu, stdp, crd_ref[:, :], jnp.zeros((nrows, d - 4), jnp.float32)],
        axis=1)
    t_ref[:, :] = jnp.concatenate([lnx, s1, s2, tail], axis=1)


def _rsqrt_nr(s):
    y = lax.bitcast_convert_type(
        jnp.int32(0x5F3759DF) - (lax.bitcast_convert_type(s, jnp.int32) >> 1),
        jnp.float32)
    for _ in range(3):
        y = y * (1.5 - 0.5 * s * y * y)
    return y


def _sc_body(deg, d, npw, t_hbm, col_hbm, crd_hbm, r_hbm,
             idx_v, rows0, rows1, crd_v, out0, out1,
             gsem0, gsem1, wsem0, wsem1):
    w = lax.axis_index("c") * 16 + lax.axis_index("s")
    nbase = w * npw
    epw = npw * deg
    gb = GRP * deg
    pltpu.sync_copy(col_hbm.at[pl.ds(nbase * deg, epw)],
                    idx_v.at[pl.ds(0, epw)])
    pltpu.sync_copy(crd_hbm.at[pl.ds(nbase * 8, npw * 8)],
                    crd_v.at[pl.ds(0, npw * 8)])
    zero16 = jnp.zeros((L,), jnp.float32)
    for i in range(gb // L):
        idx_v[pl.ds(epw + i * L, L)] = jnp.zeros((L,), jnp.int32)

    dscale = jnp.float32(-ALPHA / (RADIUS + 1e-8))
    c_mu = 3 * d
    gsems = (gsem0, gsem1)
    wsems = (wsem0, wsem1)
    rows_bufs = (rows0, rows1)
    out_bufs = (out0, out1)

    def g_start(g, buf):
        pltpu.async_copy(t_hbm.at[idx_v.at[pl.ds(g * gb, gb)]],
                         rows_bufs[buf], gsems[buf])

    def g_wait(g, buf):
        pltpu.make_async_copy(t_hbm.at[idx_v.at[pl.ds(g * gb, gb)]],
                              rows_bufs[buf], gsems[buf]).wait()

    def w_start(g, slot):
        pltpu.async_copy(out_bufs[slot],
                         r_hbm.at[pl.ds(nbase + g * GRP, GRP)], wsems[slot])

    def w_wait(slot):
        pltpu.make_async_copy(out_bufs[slot],
                              r_hbm.at[pl.ds(nbase, GRP)], wsems[slot]).wait()

    def compute(g, buf, slot):
        rows_v = rows_bufs[buf]
        out_v = out_bufs[slot]
        for t in range(GRP):
            nl = g * GRP + t
            own = crd_v[pl.ds(nl * 8, L)]
            cxn = jnp.full((L,), own[0], jnp.float32)
            cyn = jnp.full((L,), own[1], jnp.float32)
            for j in range(5 * d // L):
                out_v[t, pl.ds(j * L, L)] = zero16

            def k_body(k, musum):
                r = t * deg + k
                tail = rows_v[r, pl.ds(c_mu, L)]
                stk = jnp.full((L,), tail[1], jnp.float32)
                cx = jnp.full((L,), tail[2], jnp.float32)
                cy = jnp.full((L,), tail[3], jnp.float32)
                d2 = (cxn - cx) * (cxn - cx) + (cyn - cy) * (cyn - cy)
                s = jnp.maximum(d2, 1e-30)
                dist = s * _rsqrt_nr(s)
                dwk = jnp.exp(dist * dscale)
                for j in range(d // L):
                    nb = rows_v[r, pl.ds(j * L, L)]
                    s1 = rows_v[r, pl.ds(d + j * L, L)]
                    s2 = rows_v[r, pl.ds(2 * d + j * L, L)]
                    e1 = jnp.exp(s1 * dwk)
                    e2 = jnp.exp(s2 * dwk)
                    plsc.addupdate(out_v.at[t, pl.ds(j * L, L)], nb * stk)
                    plsc.addupdate(out_v.at[t, pl.ds(d + j * L, L)], e1)
                    plsc.addupdate(out_v.at[t, pl.ds(2 * d + j * L, L)],
                                   e1 * nb)
                    plsc.addupdate(out_v.at[t, pl.ds(3 * d + j * L, L)],
                                   e2)
                    plsc.addupdate(out_v.at[t, pl.ds(4 * d + j * L, L)],
                                   e2 * nb)
                return musum + tail[0]

            musum = lax.fori_loop(0, deg, k_body, jnp.float32(0.0))
            musum_v = jnp.full((L,), musum, jnp.float32)
            for j in range(d // L):
                plsc.addupdate(out_v.at[t, pl.ds(j * L, L)], musum_v)

    ngroups = npw // GRP
    g_start(0, 0)

    def pair_body(p, carry):
        g0 = 2 * p
        g_start(g0 + 1, 1)
        g_wait(g0, 0)

        @pl.when(p > 0)
        def _():
            w_wait(0)

        compute(g0, 0, 0)
        w_start(g0, 0)

        g_start(g0 + 2, 0)
        g_wait(g0 + 1, 1)

        @pl.when(p > 0)
        def _():
            w_wait(1)

        compute(g0 + 1, 1, 1)
        w_start(g0 + 1, 1)
        return carry

    lax.fori_loop(0, ngroups // 2, pair_body, 0)
    g_wait(ngroups, 0)
    w_wait(0)
    w_wait(1)


def _stage3_body(r_ref, g1_ref, b1_ref, ws1_ref, bs1_ref,
                 g2_ref, b2_ref, ws2_ref, bs2_ref, wr_ref, br_ref, out_ref):
    d = ws1_ref.shape[0]
    rb = r_ref[:, :]
    agg = rb[:, 0:d]
    den1 = rb[:, d:2 * d]
    m1 = rb[:, 2 * d:3 * d]
    den2 = rb[:, 3 * d:4 * d]
    m2 = rb[:, 4 * d:5 * d]

    def ln(v, g, b):
        mu = jnp.mean(v, axis=1, keepdims=True)
        var = jnp.mean((v - mu) ** 2, axis=1, keepdims=True)
        return (v - mu) * lax.rsqrt(var + LN_EPS) * g + b

    t1 = ln(agg, g1_ref[0, :], b1_ref[0, :])
    ss1 = lax.dot_general(t1, ws1_ref[:, :], (((1,), (1,)), ((), ())),
                          preferred_element_type=jnp.float32) + bs1_ref[0, :]
    es1 = jnp.exp(ss1)
    dtot1 = es1 + den1
    num1 = g1_ref[0, :] * m1 + b1_ref[0, :] * den1
    x1 = _leaky((BETA1 * es1 * t1 + (1.0 - BETA1) * num1) / dtot1)

    t2 = ln(x1, g2_ref[0, :], b2_ref[0, :])
    ss2 = lax.dot_general(t2, ws2_ref[:, :], (((1,), (1,)), ((), ())),
                          preferred_element_type=jnp.float32) + bs2_ref[0, :]
    es2 = jnp.exp(ss2)
    dtot2 = es2 + den2
    num2 = g2_ref[0, :] * m2 + b2_ref[0, :] * den2
    x2 = _leaky((BETA2 * es2 * t2 + (1.0 - BETA2) * num2) / dtot2)

    out_ref[:, :] = lax.dot_general(x2, wr_ref[:, :], (((1,), (1,)), ((), ())),
                                    preferred_element_type=jnp.float32) + br_ref[0, :]


def kernel(x, edge_index, spatial_coords, ln1_g, ln1_b, W_self1, b_self1,
           W_nei1, b_nei1, ln2_g, ln2_b, W_self2, b_self2, W_nei2, b_nei2,
           W_red, b_red):
    n, d = x.shape
    deg = edge_index.shape[1] // n
    z = W_red.shape[0]
    tw = 4 * d
    row = lambda v: v.reshape(1, -1)

    b1blk = 1000
    t_tab = pl.pallas_call(
        _stage1_body,
        grid=(n // b1blk,),
        in_specs=[
            pl.BlockSpec((b1blk, d), lambda i: (i, 0)),
            pl.BlockSpec((b1blk, 2), lambda i: (i, 0)),
            pl.BlockSpec((1, d), lambda i: (0, 0)),
            pl.BlockSpec((1, d), lambda i: (0, 0)),
            pl.BlockSpec((d, d), lambda i: (0, 0)),
            pl.BlockSpec((1, d), lambda i: (0, 0)),
            pl.BlockSpec((1, d), lambda i: (0, 0)),
            pl.BlockSpec((1, d), lambda i: (0, 0)),
            pl.BlockSpec((d, d), lambda i: (0, 0)),
            pl.BlockSpec((1, d), lambda i: (0, 0)),
        ],
        out_specs=pl.BlockSpec((b1blk, tw), lambda i: (i, 0)),
        out_shape=jax.ShapeDtypeStruct((n, tw), jnp.float32),
    )(x, spatial_coords, row(ln1_g), row(ln1_b), W_nei1, row(b_nei1),
      row(ln2_g), row(ln2_b), W_nei2, row(b_nei2))

    npw = -(-n // (NW * 64)) * 64
    npad = NW * npw
    col = edge_index[1]
    col_pad = jnp.pad(col, (0, (npad - n) * deg))
    crd_pad = jnp.zeros((npad, 8), jnp.float32).at[:n, :2].set(
        spatial_coords).reshape(-1)

    mesh = plsc.VectorSubcoreMesh(core_axis_name="c", subcore_axis_name="s",
                                  num_cores=2, num_subcores=16)
    sc = pl.kernel(
        functools.partial(_sc_body, deg, d, npw),
        out_type=jax.ShapeDtypeStruct((npad, 5 * d), jnp.float32),
        mesh=mesh,
        scratch_types=[
            pltpu.VMEM(((npw + GRP) * deg,), jnp.int32),
            pltpu.VMEM((GRP * deg, tw), jnp.float32),
            pltpu.VMEM((GRP * deg, tw), jnp.float32),
            pltpu.VMEM((npw * 8 + 8,), jnp.float32),
            pltpu.VMEM((GRP, 5 * d), jnp.float32),
            pltpu.VMEM((GRP, 5 * d), jnp.float32),
            pltpu.SemaphoreType.DMA,
            pltpu.SemaphoreType.DMA,
            pltpu.SemaphoreType.DMA,
            pltpu.SemaphoreType.DMA,
        ],
    )
    r_acc = sc(t_tab, col_pad, crd_pad)

    b3blk = 1024
    out = pl.pallas_call(
        _stage3_body,
        grid=(npad // b3blk,),
        in_specs=[
            pl.BlockSpec((b3blk, 5 * d), lambda i: (i, 0)),
            pl.BlockSpec((1, d), lambda i: (0, 0)),
            pl.BlockSpec((1, d), lambda i: (0, 0)),
            pl.BlockSpec((d, d), lambda i: (0, 0)),
            pl.BlockSpec((1, d), lambda i: (0, 0)),
            pl.BlockSpec((1, d), lambda i: (0, 0)),
            pl.BlockSpec((1, d), lambda i: (0, 0)),
            pl.BlockSpec((d, d), lambda i: (0, 0)),
            pl.BlockSpec((1, d), lambda i: (0, 0)),
            pl.BlockSpec((z, d), lambda i: (0, 0)),
            pl.BlockSpec((1, z), lambda i: (0, 0)),
        ],
        out_specs=pl.BlockSpec((b3blk, z), lambda i: (i, 0)),
        out_shape=jax.ShapeDtypeStruct((npad, z), jnp.float32),
    )(r_acc, row(ln1_g), row(ln1_b), W_self1, row(b_self1),
      row(ln2_g), row(ln2_b), W_self2, row(b_self2), W_red, row(b_red))
    return out[:n]


# speedup vs baseline: 2.2884x; 1.0393x over previous
import functools

import jax
import jax.numpy as jnp
from jax import lax
from jax.experimental import pallas as pl
from jax.experimental.pallas import tpu as pltpu
from jax.experimental.pallas import tpu_sc as plsc

RADIUS = 1.0
ALPHA = 1.0
BETA1 = 0.5
BETA2 = 0.5
NEG_SLOPE = 0.01
LN_EPS = 1e-5

L = 16
NW = 32
GRP = 2


def _leaky(v):
    return jnp.where(v >= 0, v, NEG_SLOPE * v)


def _stage1_body(x_ref, crd_ref, g1_ref, b1_ref, wn1_ref, bn1_ref,
                 g2_ref, b2_ref, wn2_ref, bn2_ref, t_ref):
    xb = x_ref[:, :]
    d = xb.shape[1]
    mu = jnp.mean(xb, axis=1, keepdims=True)
    var = jnp.mean((xb - mu) ** 2, axis=1, keepdims=True)
    stdp = jnp.sqrt(var + LN_EPS)
    lnx = (xb - mu) / stdp
    ln1x = lnx * g1_ref[0, :] + b1_ref[0, :]
    ln2x = lnx * g2_ref[0, :] + b2_ref[0, :]
    s1 = lax.dot_general(ln1x, wn1_ref[:, :], (((1,), (1,)), ((), ())),
                         preferred_element_type=jnp.float32) + bn1_ref[0, :]
    s2 = lax.dot_general(ln2x, wn2_ref[:, :], (((1,), (1,)), ((), ())),
                         preferred_element_type=jnp.float32) + bn2_ref[0, :]
    nrows = xb.shape[0]
    tail = jnp.concatenate(
        [mu, stdp, crd_ref[:, :], jnp.zeros((nrows, d - 4), jnp.float32)],
        axis=1)
    t_ref[:, :] = jnp.concatenate([lnx, s1, s2, tail], axis=1)


def _rsqrt_nr(s):
    y = lax.bitcast_convert_type(
        jnp.int32(0x5F3759DF) - (lax.bitcast_convert_type(s, jnp.int32) >> 1),
        jnp.float32)
    for _ in range(2):
        y = y * (1.5 - 0.5 * s * y * y)
    return y


def _sc_body(deg, d, npw, t_hbm, col_hbm, crd_hbm, r_hbm,
             idx_v, rows0, rows1, crd_v, out0, out1,
             gsem0, gsem1, wsem0, wsem1):
    w = lax.axis_index("c") * 16 + lax.axis_index("s")
    nbase = w * npw
    epw = npw * deg
    gb = GRP * deg
    pltpu.sync_copy(col_hbm.at[pl.ds(nbase * deg, epw)],
                    idx_v.at[pl.ds(0, epw)])
    pltpu.sync_copy(crd_hbm.at[pl.ds(nbase * 8, npw * 8)],
                    crd_v.at[pl.ds(0, npw * 8)])
    zero16 = jnp.zeros((L,), jnp.float32)
    for i in range(gb // L):
        idx_v[pl.ds(epw + i * L, L)] = jnp.zeros((L,), jnp.int32)

    dscale = jnp.float32(-ALPHA / (RADIUS + 1e-8))
    c_mu = 3 * d
    gsems = (gsem0, gsem1)
    wsems = (wsem0, wsem1)
    rows_bufs = (rows0, rows1)
    out_bufs = (out0, out1)

    def g_start(g, buf):
        pltpu.async_copy(t_hbm.at[idx_v.at[pl.ds(g * gb, gb)]],
                         rows_bufs[buf], gsems[buf])

    def g_wait(g, buf):
        pltpu.make_async_copy(t_hbm.at[idx_v.at[pl.ds(g * gb, gb)]],
                              rows_bufs[buf], gsems[buf]).wait()

    def w_start(g, slot):
        pltpu.async_copy(out_bufs[slot],
                         r_hbm.at[pl.ds(nbase + g * GRP, GRP)], wsems[slot])

    def w_wait(slot):
        pltpu.make_async_copy(out_bufs[slot],
                              r_hbm.at[pl.ds(nbase, GRP)], wsems[slot]).wait()

    def compute(g, buf, slot):
        rows_v = rows_bufs[buf]
        out_v = out_bufs[slot]
        cxn = []
        cyn = []
        for t in range(GRP):
            nl = g * GRP + t
            own = crd_v[pl.ds(nl * 8, L)]
            cxn.append(jnp.full((L,), own[0], jnp.float32))
            cyn.append(jnp.full((L,), own[1], jnp.float32))
            for j in range(5 * d // L):
                out_v[t, pl.ds(j * L, L)] = zero16

        def one_k(k, t, musum):
            r = t * deg + k
            tail = rows_v[r, pl.ds(c_mu, L)]
            stk = jnp.full((L,), tail[1], jnp.float32)
            cx = jnp.full((L,), tail[2], jnp.float32)
            cy = jnp.full((L,), tail[3], jnp.float32)
            d2 = (cxn[t] - cx) * (cxn[t] - cx) + (cyn[t] - cy) * (cyn[t] - cy)
            s = jnp.maximum(d2, 1e-30)
            dist = s * _rsqrt_nr(s)
            dwk = jnp.exp(dist * dscale)
            for j in range(d // L):
                nb = rows_v[r, pl.ds(j * L, L)]
                s1 = rows_v[r, pl.ds(d + j * L, L)]
                s2 = rows_v[r, pl.ds(2 * d + j * L, L)]
                e1 = jnp.exp(s1 * dwk)
                e2 = jnp.exp(s2 * dwk)
                plsc.addupdate(out_v.at[t, pl.ds(j * L, L)], nb * stk)
                plsc.addupdate(out_v.at[t, pl.ds(d + j * L, L)], e1)
                plsc.addupdate(out_v.at[t, pl.ds(2 * d + j * L, L)], e1 * nb)
                plsc.addupdate(out_v.at[t, pl.ds(3 * d + j * L, L)], e2)
                plsc.addupdate(out_v.at[t, pl.ds(4 * d + j * L, L)], e2 * nb)
            return musum + tail[0]

        def k_body(i, carry):
            mus = list(carry)
            for u in range(2):
                k = 2 * i + u
                for t in range(GRP):
                    mus[t] = one_k(k, t, mus[t])
            return tuple(mus)

        mus = lax.fori_loop(0, deg // 2, k_body,
                            tuple(jnp.float32(0.0) for _ in range(GRP)))
        for t in range(GRP):
            musum_v = jnp.full((L,), mus[t], jnp.float32)
            for j in range(d // L):
                plsc.addupdate(out_v.at[t, pl.ds(j * L, L)], musum_v)

    ngroups = npw // GRP
    g_start(0, 0)

    def pair_body(p, carry):
        g0 = 2 * p
        g_start(g0 + 1, 1)
        g_wait(g0, 0)

        @pl.when(p > 0)
        def _():
            w_wait(0)

        compute(g0, 0, 0)
        w_start(g0, 0)

        g_start(g0 + 2, 0)
        g_wait(g0 + 1, 1)

        @pl.when(p > 0)
        def _():
            w_wait(1)

        compute(g0 + 1, 1, 1)
        w_start(g0 + 1, 1)
        return carry

    lax.fori_loop(0, ngroups // 2, pair_body, 0)
    g_wait(ngroups, 0)
    w_wait(0)
    w_wait(1)


def _stage3_body(r_ref, g1_ref, b1_ref, ws1_ref, bs1_ref,
                 g2_ref, b2_ref, ws2_ref, bs2_ref, wr_ref, br_ref, out_ref):
    d = ws1_ref.shape[0]
    rb = r_ref[:, :]
    agg = rb[:, 0:d]
    den1 = rb[:, d:2 * d]
    m1 = rb[:, 2 * d:3 * d]
    den2 = rb[:, 3 * d:4 * d]
    m2 = rb[:, 4 * d:5 * d]

    def ln(v, g, b):
        mu = jnp.mean(v, axis=1, keepdims=True)
        var = jnp.mean((v - mu) ** 2, axis=1, keepdims=True)
        return (v - mu) * lax.rsqrt(var + LN_EPS) * g + b

    t1 = ln(agg, g1_ref[0, :], b1_ref[0, :])
    ss1 = lax.dot_general(t1, ws1_ref[:, :], (((1,), (1,)), ((), ())),
                          preferred_element_type=jnp.float32) + bs1_ref[0, :]
    es1 = jnp.exp(ss1)
    dtot1 = es1 + den1
    num1 = g1_ref[0, :] * m1 + b1_ref[0, :] * den1
    x1 = _leaky((BETA1 * es1 * t1 + (1.0 - BETA1) * num1) / dtot1)

    t2 = ln(x1, g2_ref[0, :], b2_ref[0, :])
    ss2 = lax.dot_general(t2, ws2_ref[:, :], (((1,), (1,)), ((), ())),
                          preferred_element_type=jnp.float32) + bs2_ref[0, :]
    es2 = jnp.exp(ss2)
    dtot2 = es2 + den2
    num2 = g2_ref[0, :] * m2 + b2_ref[0, :] * den2
    x2 = _leaky((BETA2 * es2 * t2 + (1.0 - BETA2) * num2) / dtot2)

    out_ref[:, :] = lax.dot_general(x2, wr_ref[:, :], (((1,), (1,)), ((), ())),
                                    preferred_element_type=jnp.float32) + br_ref[0, :]


def kernel(x, edge_index, spatial_coords, ln1_g, ln1_b, W_self1, b_self1,
           W_nei1, b_nei1, ln2_g, ln2_b, W_self2, b_self2, W_nei2, b_nei2,
           W_red, b_red):
    n, d = x.shape
    deg = edge_index.shape[1] // n
    z = W_red.shape[0]
    tw = 4 * d
    row = lambda v: v.reshape(1, -1)

    b1blk = 1000
    t_tab = pl.pallas_call(
        _stage1_body,
        grid=(n // b1blk,),
        in_specs=[
            pl.BlockSpec((b1blk, d), lambda i: (i, 0)),
            pl.BlockSpec((b1blk, 2), lambda i: (i, 0)),
            pl.BlockSpec((1, d), lambda i: (0, 0)),
            pl.BlockSpec((1, d), lambda i: (0, 0)),
            pl.BlockSpec((d, d), lambda i: (0, 0)),
            pl.BlockSpec((1, d), lambda i: (0, 0)),
            pl.BlockSpec((1, d), lambda i: (0, 0)),
            pl.BlockSpec((1, d), lambda i: (0, 0)),
            pl.BlockSpec((d, d), lambda i: (0, 0)),
            pl.BlockSpec((1, d), lambda i: (0, 0)),
        ],
        out_specs=pl.BlockSpec((b1blk, tw), lambda i: (i, 0)),
        out_shape=jax.ShapeDtypeStruct((n, tw), jnp.float32),
    )(x, spatial_coords, row(ln1_g), row(ln1_b), W_nei1, row(b_nei1),
      row(ln2_g), row(ln2_b), W_nei2, row(b_nei2))

    npw = -(-n // (NW * 64)) * 64
    npad = NW * npw
    col = edge_index[1]
    col_pad = jnp.pad(col, (0, (npad - n) * deg))
    crd_pad = jnp.zeros((npad, 8), jnp.float32).at[:n, :2].set(
        spatial_coords).reshape(-1)

    mesh = plsc.VectorSubcoreMesh(core_axis_name="c", subcore_axis_name="s",
                                  num_cores=2, num_subcores=16)
    sc = pl.kernel(
        functools.partial(_sc_body, deg, d, npw),
        out_type=jax.ShapeDtypeStruct((npad, 5 * d), jnp.float32),
        mesh=mesh,
        scratch_types=[
            pltpu.VMEM(((npw + GRP) * deg,), jnp.int32),
            pltpu.VMEM((GRP * deg, tw), jnp.float32),
            pltpu.VMEM((GRP * deg, tw), jnp.float32),
            pltpu.VMEM((npw * 8 + 8,), jnp.float32),
            pltpu.VMEM((GRP, 5 * d), jnp.float32),
            pltpu.VMEM((GRP, 5 * d), jnp.float32),
            pltpu.SemaphoreType.DMA,
            pltpu.SemaphoreType.DMA,
            pltpu.SemaphoreType.DMA,
            pltpu.SemaphoreType.DMA,
        ],
    )
    r_acc = sc(t_tab, col_pad, crd_pad)

    b3blk = 1024
    out = pl.pallas_call(
        _stage3_body,
        grid=(npad // b3blk,),
        in_specs=[
            pl.BlockSpec((b3blk, 5 * d), lambda i: (i, 0)),
            pl.BlockSpec((1, d), lambda i: (0, 0)),
            pl.BlockSpec((1, d), lambda i: (0, 0)),
            pl.BlockSpec((d, d), lambda i: (0, 0)),
            pl.BlockSpec((1, d), lambda i: (0, 0)),
            pl.BlockSpec((1, d), lambda i: (0, 0)),
            pl.BlockSpec((1, d), lambda i: (0, 0)),
            pl.BlockSpec((d, d), lambda i: (0, 0)),
            pl.BlockSpec((1, d), lambda i: (0, 0)),
            pl.BlockSpec((z, d), lambda i: (0, 0)),
            pl.BlockSpec((1, z), lambda i: (0, 0)),
        ],
        out_specs=pl.BlockSpec((b3blk, z), lambda i: (i, 0)),
        out_shape=jax.ShapeDtypeStruct((npad, z), jnp.float32),
    )(r_acc, row(ln1_g), row(ln1_b), W_self1, row(b_self1),
      row(ln2_g), row(ln2_b), W_self2, row(b_self2), W_red, row(b_red))
    return out[:n]
